# baseline (speedup 1.0000x reference)
import functools

import jax
import jax.numpy as jnp
from jax import lax
from jax.experimental import pallas as pl
from jax.experimental.pallas import tpu as pltpu
from jax.experimental.pallas import tpu_sc as plsc

B = 128
N = 35709
K = 64
C = 17
F = 51
NLM = C + F
NW = 32
BPW = B // NW
NIDX = 1152
NCH = 27
OUTW = NLM * 3


def _sc_body(tabf_hbm, idx3base_hbm, out_hbm,
             idx3_base_v, idx3_v, buf, srcrow, outacc, sem):
    nc = 2
    wid = lax.axis_index("s") * nc + lax.axis_index("c")
    iota = lax.iota(jnp.int32, 16)

    pltpu.sync_copy(idx3base_hbm, idx3_base_v)

    def batch_body(lb, carry):
        b = wid * BPW + lb
        off = b * (3 * N)

        def add_body(s, c2):
            j = s // 8
            t = s % 8
            idx3_v[j, pl.ds(t * 16, 16)] = (
                idx3_base_v[j, pl.ds(t * 16, 16)] + off)
            return c2
        lax.fori_loop(0, NCH * 8, add_body, 0)

        copies = [
            pltpu.async_copy(tabf_hbm.at[idx3_v.at[j]],
                             buf.at[pl.ds(j * 128, 128)], sem)
            for j in range(NCH)
        ]
        for cp in copies:
            cp.wait()

        for s in range(5):
            srcrow[pl.ds(s * 16, 16)] = (s * 16 + (K * C - C)) + iota

        def col_body(c, c2):
            comp = jnp.where(c == 8, 1, 0)
            sgn = jnp.where(c < 8, 1.0, -1.0)
            base = c * K

            def chunk(q, st):
                bk, br = st
                rows = base + q * 16 + iota
                v = plsc.load_gather(buf, [rows * 3 + comp])
                key = v * sgn
                upd = key < bk
                return (jnp.where(upd, key, bk), jnp.where(upd, rows, br))

            best_key, best_row = lax.fori_loop(
                0, K // 16, chunk,
                (jnp.full((16,), jnp.inf, jnp.float32),
                 jnp.zeros((16,), jnp.int32)))
            m = jnp.min(best_key)
            win = jnp.min(jnp.where(best_key == m, best_row,
                                    jnp.int32(2 ** 30)))
            plsc.store_scatter(srcrow, [jnp.full((16,), c, jnp.int32)],
                               jnp.full((16,), win, jnp.int32),
                               mask=iota == 0)
            return c2
        lax.fori_loop(0, C, col_body, 0)

        obase = lb * OUTW

        def slot_body(s, c2):
            p = s * 16 + iota
            i = p // 3
            d = p - i * 3
            sr = plsc.load_gather(srcrow, [i])
            v = plsc.load_gather(buf, [sr * 3 + d])
            plsc.store_scatter(outacc, [obase + p], v)
            return c2
        lax.fori_loop(0, 13, slot_body, 0)
        return carry

    lax.fori_loop(0, BPW, batch_body, 0)

    pltpu.sync_copy(outacc.at[pl.ds(0, BPW * OUTW)],
                    out_hbm.at[pl.ds(wid * BPW * OUTW, BPW * OUTW)])


def kernel(batch_cam_vps, contour_idx, inface_idx):
    tabf = batch_cam_vps.reshape(B * N * 3)
    ci = contour_idx.astype(jnp.int32)
    base_rows = jnp.concatenate([
        ci.T.reshape(-1),
        inface_idx.astype(jnp.int32),
        jnp.zeros((NIDX - K * C - F,), jnp.int32),
    ])
    idx3 = (3 * base_rows[:, None] + jnp.arange(3, dtype=jnp.int32)
            ).reshape(NCH, 128)

    mesh = plsc.VectorSubcoreMesh(core_axis_name="c", subcore_axis_name="s")
    run = functools.partial(
        pl.kernel,
        out_type=jax.ShapeDtypeStruct((B * NLM * 3,), jnp.float32),
        mesh=mesh,
        compiler_params=pltpu.CompilerParams(needs_layout_passes=False,
                                             use_tc_tiling_on_sc=False),
        scratch_types=[
            pltpu.VMEM((NCH, 128), jnp.int32),
            pltpu.VMEM((NCH, 128), jnp.int32),
            pltpu.VMEM((NIDX * 3,), jnp.float32),
            pltpu.VMEM((80,), jnp.int32),
            pltpu.VMEM((BPW * OUTW + 16,), jnp.float32),
            pltpu.SemaphoreType.DMA,
        ],
    )(_sc_body)
    out = run(tabf, idx3)
    return out.reshape(B, NLM, 3)


# speedup vs baseline: 27.2737x; 27.2737x over previous
import functools

import jax
import jax.numpy as jnp
from jax import lax
from jax.experimental import pallas as pl
from jax.experimental.pallas import tpu as pltpu
from jax.experimental.pallas import tpu_sc as plsc

B = 128
N = 35709
K = 64
C = 17
F = 51
NLM = C + F
NW = 32
BPW = B // NW
NIDX = 1152
NCH = 27
OUTW = NLM * 3


def _sc_body(tabf_hbm, idx3base_hbm, out_hbm,
             idx3_base_v, idx3_v, buf, srcrow, outacc, sem):
    nc = 2
    wid = lax.axis_index("s") * nc + lax.axis_index("c")
    iota = lax.iota(jnp.int32, 16)

    pltpu.sync_copy(idx3base_hbm, idx3_base_v)

    def batch_body(lb, carry):
        b = wid * BPW + lb
        off = b * N

        def add_body(s, c2):
            j = s // 8
            t = s % 8
            idx3_v[j, pl.ds(t * 16, 16)] = (
                idx3_base_v[j, pl.ds(t * 16, 16)] + off)
            return c2
        lax.fori_loop(0, NCH * 8, add_body, 0)

        copies = [
            pltpu.async_copy(tabf_hbm.at[idx3_v.at[j]],
                             buf.at[pl.ds(j * 128, 128)], sem)
            for j in range(NCH)
        ]
        for cp in copies:
            cp.wait()

        for s in range(5):
            srcrow[pl.ds(s * 16, 16)] = (s * 16 + (K * C - C)) + iota

        def col_body(c, c2):
            comp = jnp.where(c == 8, 1, 0)
            sgn = jnp.where(c < 8, 1.0, -1.0)
            base = c * K

            def chunk(q, st):
                bk, br = st
                rows = base + q * 16 + iota
                v = plsc.load_gather(buf, [rows * 3 + comp])
                key = v * sgn
                upd = key < bk
                return (jnp.where(upd, key, bk), jnp.where(upd, rows, br))

            best_key, best_row = lax.fori_loop(
                0, K // 16, chunk,
                (jnp.full((16,), jnp.inf, jnp.float32),
                 jnp.zeros((16,), jnp.int32)))
            m = jnp.min(best_key)
            win = jnp.min(jnp.where(best_key == m, best_row,
                                    jnp.int32(2 ** 30)))
            plsc.store_scatter(srcrow, [jnp.full((16,), c, jnp.int32)],
                               jnp.full((16,), win, jnp.int32),
                               mask=iota == 0)
            return c2
        lax.fori_loop(0, C, col_body, 0)

        obase = lb * OUTW

        def slot_body(s, c2):
            p = s * 16 + iota
            i = p // 3
            d = p - i * 3
            sr = plsc.load_gather(srcrow, [i])
            v = plsc.load_gather(buf, [sr * 3 + d])
            plsc.store_scatter(outacc, [obase + p], v)
            return c2
        lax.fori_loop(0, 13, slot_body, 0)
        return carry

    lax.fori_loop(0, BPW, batch_body, 0)

    pltpu.sync_copy(outacc.at[pl.ds(0, BPW * OUTW)],
                    out_hbm.at[pl.ds(wid * BPW * OUTW, BPW * OUTW)])


def kernel(batch_cam_vps, contour_idx, inface_idx):
    tabf = jnp.transpose(batch_cam_vps, (2, 0, 1)).reshape(3 * B * N)
    ci = contour_idx.astype(jnp.int32)
    base_rows = jnp.concatenate([
        ci.T.reshape(-1),
        inface_idx.astype(jnp.int32),
        jnp.zeros((NIDX - K * C - F,), jnp.int32),
    ])
    idx3 = (base_rows[:, None] + (B * N) * jnp.arange(3, dtype=jnp.int32)
            ).reshape(NCH, 128)

    mesh = plsc.VectorSubcoreMesh(core_axis_name="c", subcore_axis_name="s")
    run = functools.partial(
        pl.kernel,
        out_type=jax.ShapeDtypeStruct((B * NLM * 3,), jnp.float32),
        mesh=mesh,
        compiler_params=pltpu.CompilerParams(needs_layout_passes=False,
                                             use_tc_tiling_on_sc=False),
        scratch_types=[
            pltpu.VMEM((NCH, 128), jnp.int32),
            pltpu.VMEM((NCH, 128), jnp.int32),
            pltpu.VMEM((NIDX * 3,), jnp.float32),
            pltpu.VMEM((80,), jnp.int32),
            pltpu.VMEM((BPW * OUTW + 16,), jnp.float32),
            pltpu.SemaphoreType.DMA,
        ],
    )(_sc_body)
    out = run(tabf, idx3)
    return out.reshape(B, NLM, 3)


# speedup vs baseline: 102.7124x; 3.7660x over previous
import functools

import jax
import jax.numpy as jnp
from jax import lax
from jax.experimental import pallas as pl
from jax.experimental.pallas import tpu as pltpu
from jax.experimental.pallas import tpu_sc as plsc

B = 128
N = 35709
K = 64
C = 17
F = 51
NLM = C + F
NCAND = 1152
VB = 512
NB0 = 35
NB1 = 34
TAILV = 35328
TAILW = N - TAILV
OUTW = NLM * 3
EXTW = 24


def _extract_block(buf, ext, vs_v, bs_v, j, d, iota, lane_hi, b_i):
    win = bs_v[pl.ds(j, 16)]
    s_lo = win[0]
    s_hi = win[1]
    vt = j * VB
    npairs = (s_hi - s_lo + 1) // 2

    def pair(i, carry):
        cand = s_lo + 2 * i + lane_hi
        msk = cand < s_hi
        v = plsc.load_gather(vs_v, [cand], mask=msk)
        col = v - vt
        val = plsc.load_gather(buf, [b_i, col], mask=msk)
        dst = cand * EXTW + d * 8 + b_i
        plsc.store_scatter(ext, [dst], val, mask=msk)
        return carry
    lax.fori_loop(0, npairs, pair, 0)


def _sc_body(tab_hbm, vs_hbm, sp_hbm, bs_hbm, out_hbm,
             vs_v, sp_v, bs_v, buf, tbuf, ext, ext2, srcrow, outacc, shared):
    c = lax.axis_index("c")
    s = lax.axis_index("s")
    bbl = s % 8
    h = s // 8
    gbb = c * 8 + bbl
    b0 = gbb * 8
    iota = lax.iota(jnp.int32, 16)
    lane_hi = (iota >= 8).astype(jnp.int32)
    b_i = iota % 8

    pltpu.sync_copy(vs_hbm, vs_v.at[pl.ds(0, NCAND)])
    pltpu.sync_copy(sp_hbm, sp_v)
    pltpu.sync_copy(bs_hbm, bs_v.at[pl.ds(0, 80)])

    jbase = jnp.where(h == 0, 0, NB0)
    nfull = jnp.where(h == 0, NB0, NB1)

    for d in range(3):
        row = pl.multiple_of(d * 128 + b0, 8)

        def block(jj, carry, d=d, row=row):
            j = jbase + jj
            vt = pl.multiple_of(j * VB, 128)
            pltpu.sync_copy(tab_hbm.at[pl.ds(row, 8), pl.ds(vt, VB)], buf)
            _extract_block(buf, ext, vs_v, bs_v, j, d, iota, lane_hi, b_i)
            return carry
        lax.fori_loop(0, nfull, block, 0)

        @pl.when(h == 1)
        def _(d=d, row=row):
            pltpu.sync_copy(
                tab_hbm.at[pl.ds(row, 8), pl.ds(TAILV, TAILW)],
                tbuf.at[:, pl.ds(0, TAILW)])
            _extract_block(tbuf, ext, vs_v, bs_v,
                           jnp.int32(NB0 + NB1), d, iota, lane_hi, b_i)

    @pl.when(h == 1)
    def _():
        pltpu.sync_copy(ext, shared.at[bbl])
    plsc.subcore_barrier()

    @pl.when(h == 0)
    def _():
        pltpu.sync_copy(shared.at[bbl], ext2)
        n0 = bs_v[pl.ds(NB0, 16)][0]

        for ss in range(5):
            srcrow[pl.ds(ss * 16, 16)] = (ss * 16 + (K * C - C)) + iota

        def merged_gather(addr, sp):
            v0 = plsc.load_gather(ext, [addr])
            v1 = plsc.load_gather(ext2, [addr])
            return jnp.where(sp < n0, v0, v1)

        def batch_body(bi, carry):
            def col_body(cc, c2):
                comp = jnp.where(cc == 8, 1, 0)
                sgn = jnp.where(cc < 8, 1.0, -1.0)
                base = cc * K

                def chunk(q, st):
                    bk, br = st
                    slots = base + q * 16 + iota
                    sp = plsc.load_gather(sp_v, [slots])
                    val = merged_gather(sp * EXTW + comp * 8 + bi, sp)
                    key = val * sgn
                    upd = key < bk
                    return (jnp.where(upd, key, bk),
                            jnp.where(upd, slots, br))

                best_key, best_row = lax.fori_loop(
                    0, K // 16, chunk,
                    (jnp.full((16,), jnp.inf, jnp.float32),
                     jnp.zeros((16,), jnp.int32)))
                m = jnp.min(best_key)
                win = jnp.min(jnp.where(best_key == m, best_row,
                                        jnp.int32(2 ** 30)))
                plsc.store_scatter(srcrow, [jnp.full((16,), cc, jnp.int32)],
                                   jnp.full((16,), win, jnp.int32),
                                   mask=iota == 0)
                return c2
            lax.fori_loop(0, C, col_body, 0)

            def slot_body(t, c2):
                p = t * 16 + iota
                i = p // 3
                dc = p - i * 3
                slot = plsc.load_gather(srcrow, [i])
                sp = plsc.load_gather(sp_v, [slot])
                val = merged_gather(sp * EXTW + dc * 8 + bi, sp)
                plsc.store_scatter(outacc, [bi * OUTW + p], val)
                return c2
            lax.fori_loop(0, 13, slot_body, 0)
            return carry
        lax.fori_loop(0, 8, batch_body, 0)

        pltpu.sync_copy(outacc.at[pl.ds(0, 8 * OUTW)],
                        out_hbm.at[pl.ds(gbb * 8 * OUTW, 8 * OUTW)])


def kernel(batch_cam_vps, contour_idx, inface_idx):
    tab = jnp.transpose(batch_cam_vps, (2, 0, 1)).reshape(3 * B, N)

    ci = contour_idx.astype(jnp.int32)
    rows_canon = jnp.concatenate([
        ci.T.reshape(-1),
        inface_idx.astype(jnp.int32),
        jnp.zeros((NCAND - K * C - F,), jnp.int32),
    ])
    order = jnp.argsort(rows_canon)
    vsorted = rows_canon[order].astype(jnp.int32)
    sortpos = jnp.argsort(order).astype(jnp.int32)
    bnds = jnp.concatenate([
        jnp.arange(NB0 + NB1 + 1, dtype=jnp.int32) * VB,
        jnp.array([N], jnp.int32)])
    blockstart = jnp.concatenate([
        jnp.searchsorted(vsorted, bnds).astype(jnp.int32),
        jnp.zeros((80 - (NB0 + NB1 + 2),), jnp.int32)])

    mesh = plsc.VectorSubcoreMesh(core_axis_name="c", subcore_axis_name="s")
    run = functools.partial(
        pl.kernel,
        out_type=jax.ShapeDtypeStruct((B * OUTW,), jnp.float32),
        mesh=mesh,
        compiler_params=pltpu.CompilerParams(needs_layout_passes=False,
                                             use_tc_tiling_on_sc=True),
        scratch_types=[
            pltpu.VMEM((NCAND + 32,), jnp.int32),
            pltpu.VMEM((NCAND,), jnp.int32),
            pltpu.VMEM((96,), jnp.int32),
            pltpu.VMEM((8, VB), jnp.float32),
            pltpu.VMEM((8, TAILW), jnp.float32),
            pltpu.VMEM((NCAND * EXTW,), jnp.float32),
            pltpu.VMEM((NCAND * EXTW,), jnp.float32),
            pltpu.VMEM((80,), jnp.int32),
            pltpu.VMEM((8 * OUTW + 32,), jnp.float32),
            pltpu.VMEM_SHARED((8, NCAND * EXTW), jnp.float32),
        ],
    )(_sc_body)
    out = run(tab, vsorted, sortpos, blockstart)
    return out.reshape(B, NLM, 3)


# speedup vs baseline: 207.0802x; 2.0161x over previous
import functools

import jax
import jax.numpy as jnp
from jax import lax
from jax.experimental import pallas as pl
from jax.experimental.pallas import tpu as pltpu
from jax.experimental.pallas import tpu_sc as plsc

B = 128
N = 35709
K = 64
C = 17
F = 51
NLM = C + F
NCAND = 1152
VB = 1792
NBF = 19
TAILV = NBF * VB
TAILW = N - TAILV
NB0 = 10
NB1 = 9
OUTW = NLM * 3
EXTW = 24


def _extract_block(buf, ext, vs_v, bs_v, j, d, iota, lane_hi, b_i):
    win = bs_v[pl.ds(j, 16)]
    s_lo = win[0]
    s_hi = win[1]
    vt = j * VB
    npairs = (s_hi - s_lo + 1) // 2

    def pair(i, carry):
        cand = s_lo + 2 * i + lane_hi
        msk = cand < s_hi
        v = plsc.load_gather(vs_v, [cand], mask=msk)
        col = v - vt
        val = plsc.load_gather(buf, [b_i, col], mask=msk)
        dst = cand * EXTW + d * 8 + b_i
        plsc.store_scatter(ext, [dst], val, mask=msk)
        return carry
    lax.fori_loop(0, npairs, pair, 0)


def _sc_body(tab_hbm, vs_hbm, sp_hbm, bs_hbm, out_hbm,
             vs_v, sp_v, bs_v, bufa, bufb, tbuf, ext, ext2, srcrow, outacc,
             shared, sema, semb):
    c = lax.axis_index("c")
    s = lax.axis_index("s")
    bbl = s % 8
    h = s // 8
    gbb = c * 8 + bbl
    b0 = gbb * 8
    iota = lax.iota(jnp.int32, 16)
    lane_hi = (iota >= 8).astype(jnp.int32)
    b_i = iota % 8

    pltpu.sync_copy(vs_hbm, vs_v.at[pl.ds(0, NCAND)])
    pltpu.sync_copy(sp_hbm, sp_v)
    pltpu.sync_copy(bs_hbm, bs_v)

    @pl.when(h == 1)
    def _():
        for d in range(3):
            row = pl.multiple_of(d * 128 + b0, 8)
            pltpu.sync_copy(
                tab_hbm.at[pl.ds(row, 8), pl.ds(TAILV, TAILW)], tbuf)
            _extract_block(tbuf, ext, vs_v, bs_v,
                           jnp.int32(NBF), d, iota, lane_hi, b_i)

    jb = jnp.where(h == 0, 0, NB0)
    nb = jnp.where(h == 0, NB0, NB1)
    total = 3 * nb

    def blk_slice(t):
        d = t // nb
        j = jb + t % nb
        row = pl.multiple_of(d * 128 + b0, 8)
        vt = pl.multiple_of((jb + t % nb) * VB, 128)
        return tab_hbm.at[pl.ds(row, 8), pl.ds(vt, VB)], d, j

    def issue(t, buf, sem):
        @pl.when(t < total)
        def _():
            src, _, _ = blk_slice(t)
            pltpu.async_copy(src, buf, sem)

    def drain_extract(t, buf, sem):
        @pl.when(t < total)
        def _():
            src, d, j = blk_slice(t)
            pltpu.make_async_copy(src, buf, sem).wait()
            _extract_block(buf, ext, vs_v, bs_v, j, d, iota, lane_hi, b_i)

    issue(jnp.int32(0), bufa, sema)

    def pipe(i, carry):
        t0 = 2 * i
        t1 = t0 + 1
        issue(t1, bufb, semb)
        drain_extract(t0, bufa, sema)
        issue(t0 + 2, bufa, sema)
        drain_extract(t1, bufb, semb)
        return carry
    lax.fori_loop(0, 15, pipe, 0)

    @pl.when(h == 1)
    def _():
        pltpu.sync_copy(ext, shared.at[bbl])
    plsc.subcore_barrier()

    @pl.when(h == 0)
    def _():
        pltpu.sync_copy(shared.at[bbl], ext2)
        n0 = bs_v[pl.ds(NB0, 16)][0]

        for ss in range(5):
            srcrow[pl.ds(ss * 16, 16)] = (ss * 16 + (K * C - C)) + iota

        def merged_gather(addr, sp):
            v0 = plsc.load_gather(ext, [addr])
            v1 = plsc.load_gather(ext2, [addr])
            return jnp.where(sp < n0, v0, v1)

        def batch_body(bi, carry):
            def col_body(cc, c2):
                comp = jnp.where(cc == 8, 1, 0)
                sgn = jnp.where(cc < 8, 1.0, -1.0)
                base = cc * K

                def chunk(q, st):
                    bk, br = st
                    slots = base + q * 16 + iota
                    sp = plsc.load_gather(sp_v, [slots])
                    val = merged_gather(sp * EXTW + comp * 8 + bi, sp)
                    key = val * sgn
                    upd = key < bk
                    return (jnp.where(upd, key, bk),
                            jnp.where(upd, slots, br))

                best_key, best_row = lax.fori_loop(
                    0, K // 16, chunk,
                    (jnp.full((16,), jnp.inf, jnp.float32),
                     jnp.zeros((16,), jnp.int32)))
                m = jnp.min(best_key)
                win = jnp.min(jnp.where(best_key == m, best_row,
                                        jnp.int32(2 ** 30)))
                plsc.store_scatter(srcrow, [jnp.full((16,), cc, jnp.int32)],
                                   jnp.full((16,), win, jnp.int32),
                                   mask=iota == 0)
                return c2
            lax.fori_loop(0, C, col_body, 0)

            def slot_body(t, c2):
                p = t * 16 + iota
                i = p // 3
                dc = p - i * 3
                slot = plsc.load_gather(srcrow, [i])
                sp = plsc.load_gather(sp_v, [slot])
                val = merged_gather(sp * EXTW + dc * 8 + bi, sp)
                plsc.store_scatter(outacc, [bi * OUTW + p], val)
                return c2
            lax.fori_loop(0, 13, slot_body, 0)
            return carry
        lax.fori_loop(0, 8, batch_body, 0)

        pltpu.sync_copy(outacc.at[pl.ds(0, 8 * OUTW)],
                        out_hbm.at[pl.ds(gbb * 8 * OUTW, 8 * OUTW)])


def kernel(batch_cam_vps, contour_idx, inface_idx):
    tab = jnp.transpose(batch_cam_vps, (2, 0, 1)).reshape(3 * B, N)

    ci = contour_idx.astype(jnp.int32)
    rows_canon = jnp.concatenate([
        ci.T.reshape(-1),
        inface_idx.astype(jnp.int32),
        jnp.zeros((NCAND - K * C - F,), jnp.int32),
    ])
    vsorted = jnp.sort(rows_canon)
    sortpos = jnp.argsort(jnp.argsort(rows_canon)).astype(jnp.int32)
    bnds = jnp.concatenate([
        jnp.arange(NBF + 1, dtype=jnp.int32) * VB,
        jnp.array([N], jnp.int32)])
    blockstart = jnp.concatenate([
        jnp.searchsorted(vsorted, bnds).astype(jnp.int32),
        jnp.zeros((48 - (NBF + 2),), jnp.int32)])

    mesh = plsc.VectorSubcoreMesh(core_axis_name="c", subcore_axis_name="s")
    run = functools.partial(
        pl.kernel,
        out_type=jax.ShapeDtypeStruct((B * OUTW,), jnp.float32),
        mesh=mesh,
        compiler_params=pltpu.CompilerParams(needs_layout_passes=False,
                                             use_tc_tiling_on_sc=True),
        scratch_types=[
            pltpu.VMEM((NCAND + 32,), jnp.int32),
            pltpu.VMEM((NCAND,), jnp.int32),
            pltpu.VMEM((48,), jnp.int32),
            pltpu.VMEM((8, VB), jnp.float32),
            pltpu.VMEM((8, VB), jnp.float32),
            pltpu.VMEM((8, TAILW), jnp.float32),
            pltpu.VMEM((NCAND * EXTW,), jnp.float32),
            pltpu.VMEM((NCAND * EXTW,), jnp.float32),
            pltpu.VMEM((80,), jnp.int32),
            pltpu.VMEM((8 * OUTW + 32,), jnp.float32),
            pltpu.VMEM_SHARED((8, NCAND * EXTW), jnp.float32),
            pltpu.SemaphoreType.DMA,
            pltpu.SemaphoreType.DMA,
        ],
    )(_sc_body)
    out = run(tab, vsorted.astype(jnp.int32), sortpos, blockstart)
    return out.reshape(B, NLM, 3)
